# baseline (speedup 1.0000x reference)
import jax
import jax.numpy as jnp
from jax import lax
from jax.experimental import pallas as pl
from jax.experimental.pallas import tpu as pltpu
from jax.experimental.pallas import tpu_sc as plsc

_B, _C, _H, _W = 8, 255, 76, 76
_S = _H * _W
_NC, _NS = 2, 16
_COLS = 2 * _W
_NT = _S // _COLS
_PW = (10.0 / 608.0, 16.0 / 608.0, 33.0 / 608.0)
_PH = (13.0 / 608.0, 30.0 / 608.0, 23.0 / 608.0)
_OFFS = (0, 16, 32, 48, 64, 80, 96, 112, 128, 136)


def _decode_body(x_ref, y_ref, inb, outb):
    wid = lax.axis_index("s") * _NC + lax.axis_index("c")
    b = wid // 4
    q = wid % 4
    start = jnp.where(q < 2, q * 10, 20 + (q - 2) * 9)
    trip = jnp.where(q < 2, 10, 9)
    iota = lax.iota(jnp.int32, 16)

    def tile(k, carry):
        j = start + k
        s0 = j * _COLS
        row0 = 2 * j
        pltpu.sync_copy(x_ref.at[b, :, pl.ds(s0, _COLS)], inb)

        for off in _OFFS:
            scv = off + iota
            ge = scv >= _W
            wvf = jnp.where(ge, scv - _W, scv).astype(jnp.float32)
            hvf = (jnp.full((16,), row0, jnp.int32)
                   + ge.astype(jnp.int32)).astype(jnp.float32)
            for a in range(3):
                c0 = 85 * a
                v0 = inb[c0, pl.ds(off, 16)]
                r0 = (1.0 / (1.0 + jnp.exp(-v0)) + wvf) * (1.0 / _W)
                plsc.store_scatter(outb, [scv, jnp.full((16,), c0, jnp.int32)], r0)
                v1 = inb[c0 + 1, pl.ds(off, 16)]
                r1 = (1.0 / (1.0 + jnp.exp(-v1)) + hvf) * (1.0 / _H)
                plsc.store_scatter(outb, [scv, jnp.full((16,), c0 + 1, jnp.int32)], r1)
                v2 = inb[c0 + 2, pl.ds(off, 16)]
                plsc.store_scatter(outb, [scv, jnp.full((16,), c0 + 2, jnp.int32)],
                                   _PW[a] * jnp.exp(v2))
                v3 = inb[c0 + 3, pl.ds(off, 16)]
                plsc.store_scatter(outb, [scv, jnp.full((16,), c0 + 3, jnp.int32)],
                                   _PH[a] * jnp.exp(v3))

        for a in range(3):
            base = 85 * a + 4

            def ch(i, carry2):
                c = base + i
                cvec = jnp.full((16,), c, jnp.int32)
                for off in _OFFS:
                    v = inb[c, pl.ds(off, 16)]
                    r = 1.0 / (1.0 + jnp.exp(-v))
                    plsc.store_scatter(outb, [off + iota, cvec], r)
                return carry2

            lax.fori_loop(0, 81, ch, 0)

        pltpu.sync_copy(outb, y_ref.at[b, pl.ds(s0, _COLS), :])
        return carry

    lax.fori_loop(0, trip, tile, 0)


def kernel(x):
    xr = x.reshape(_B, _C, _S)
    mesh = plsc.VectorSubcoreMesh(core_axis_name="c", subcore_axis_name="s")
    y = pl.kernel(
        _decode_body,
        out_type=jax.ShapeDtypeStruct((_B, _S, _C), jnp.float32),
        mesh=mesh,
        scratch_types=[
            pltpu.VMEM((_C, _COLS), jnp.float32),
            pltpu.VMEM((_COLS, _C), jnp.float32),
        ],
        compiler_params=pltpu.CompilerParams(
            use_tc_tiling_on_sc=False, needs_layout_passes=False),
    )(xr)
    return y.reshape(_B, _S * _C // 85, 85)


# speedup vs baseline: 1.4027x; 1.4027x over previous
import jax
import jax.numpy as jnp
from jax import lax
from jax.experimental import pallas as pl
from jax.experimental.pallas import tpu as pltpu
from jax.experimental.pallas import tpu_sc as plsc

_B, _C, _H, _W = 8, 255, 76, 76
_S = _H * _W
_NC, _NS = 2, 16
_COLS = 2 * _W
_NT = _S // _COLS
_PW = (10.0 / 608.0, 16.0 / 608.0, 33.0 / 608.0)
_PH = (13.0 / 608.0, 30.0 / 608.0, 23.0 / 608.0)
_OFFS = (0, 16, 32, 48, 64, 80, 96, 112, 128, 136)


def _decode_body(x_ref, y_ref, inb, outb):
    wid = lax.axis_index("s") * _NC + lax.axis_index("c")
    b = wid // 4
    q = wid % 4
    start = jnp.where(q < 2, q * 10, 20 + (q - 2) * 9)
    trip = jnp.where(q < 2, 10, 9)
    iota = lax.iota(jnp.int32, 16)

    def tile(k, carry):
        j = start + k
        s0 = j * _COLS
        row0 = 2 * j
        pltpu.sync_copy(x_ref.at[b, :, pl.ds(s0, _COLS)], inb)

        for off in _OFFS:
            scv = off + iota
            ge = scv >= _W
            wvf = jnp.where(ge, scv - _W, scv).astype(jnp.float32)
            hvf = (jnp.full((16,), row0, jnp.int32)
                   + ge.astype(jnp.int32)).astype(jnp.float32)
            for a in range(3):
                c0 = 85 * a
                v0 = inb[c0, pl.ds(off, 16)]
                r0 = (1.0 / (1.0 + jnp.exp(-v0)) + wvf) * (1.0 / _W)
                plsc.store_scatter(outb, [scv, jnp.full((16,), c0, jnp.int32)], r0)
                v1 = inb[c0 + 1, pl.ds(off, 16)]
                r1 = (1.0 / (1.0 + jnp.exp(-v1)) + hvf) * (1.0 / _H)
                plsc.store_scatter(outb, [scv, jnp.full((16,), c0 + 1, jnp.int32)], r1)
                v2 = inb[c0 + 2, pl.ds(off, 16)]
                plsc.store_scatter(outb, [scv, jnp.full((16,), c0 + 2, jnp.int32)],
                                   _PW[a] * jnp.exp(v2))
                v3 = inb[c0 + 3, pl.ds(off, 16)]
                plsc.store_scatter(outb, [scv, jnp.full((16,), c0 + 3, jnp.int32)],
                                   _PH[a] * jnp.exp(v3))

        @plsc.parallel_loop(0, 81, 1, unroll=2)
        def ch(i):
            for a in range(3):
                c = 85 * a + 4 + i
                cvec = jnp.full((16,), c, jnp.int32)
                for off in _OFFS:
                    v = inb[c, pl.ds(off, 16)]
                    r = 1.0 / (1.0 + jnp.exp(-v))
                    plsc.store_scatter(outb, [off + iota, cvec], r)

        pltpu.sync_copy(outb, y_ref.at[b, pl.ds(s0, _COLS), :])
        return carry

    lax.fori_loop(0, trip, tile, 0)


def kernel(x):
    xr = x.reshape(_B, _C, _S)
    mesh = plsc.VectorSubcoreMesh(core_axis_name="c", subcore_axis_name="s")
    y = pl.kernel(
        _decode_body,
        out_type=jax.ShapeDtypeStruct((_B, _S, _C), jnp.float32),
        mesh=mesh,
        scratch_types=[
            pltpu.VMEM((_C, _COLS), jnp.float32),
            pltpu.VMEM((_COLS, _C), jnp.float32),
        ],
        compiler_params=pltpu.CompilerParams(
            use_tc_tiling_on_sc=False, needs_layout_passes=False),
    )(xr)
    return y.reshape(_B, _S * _C // 85, 85)
